# baseline (speedup 1.0000x reference)
import functools

import jax
import jax.numpy as jnp
from jax import lax
from jax.experimental import pallas as pl
from jax.experimental.pallas import tpu as pltpu
from jax.experimental.pallas import tpu_sc as plsc

_NC = 2
_NS = 16
_NW = _NC * _NS
_RPW = 8
_LANES = 16


def _embed_sum_body(active, embed, idx_hbm, emb_hbm, out_hbm,
                    idx_v, rows_v, acc_v, all_v, shared, sem):
    c = lax.axis_index("c")
    s = lax.axis_index("s")
    wid = s * _NC + c
    base = wid * _RPW
    pltpu.sync_copy(idx_hbm.at[pl.ds(base, _RPW)], idx_v)
    pltpu.async_copy(emb_hbm.at[idx_v], rows_v, sem).wait()
    scale = jnp.where(wid < active, jnp.float32(1), jnp.float32(0))
    for ch in range(embed // _LANES):
        v = rows_v[0, pl.ds(ch * _LANES, _LANES)]
        for r in range(1, _RPW):
            v = v + rows_v[r, pl.ds(ch * _LANES, _LANES)]
        acc_v[pl.ds(ch * _LANES, _LANES)] = v * scale
    pltpu.sync_copy(acc_v, shared.at[s])
    plsc.subcore_barrier()

    @pl.when(s == 0)
    def _():
        pltpu.sync_copy(shared, all_v)
        for ch in range(embed // _LANES):
            v = all_v[0, pl.ds(ch * _LANES, _LANES)]
            for w in range(1, _NS):
                v = v + all_v[w, pl.ds(ch * _LANES, _LANES)]
            acc_v[pl.ds(ch * _LANES, _LANES)] = v
        pltpu.sync_copy(acc_v, out_hbm.at[c])


def _embed_sum_sc(idx_pad, emb):
    embed = emb.shape[1]
    active = idx_pad.shape[0] // _RPW
    mesh = plsc.VectorSubcoreMesh(
        core_axis_name="c", subcore_axis_name="s",
        num_cores=_NC, num_subcores=_NS)
    padded = jnp.concatenate(
        [idx_pad, jnp.zeros(((_NW * _RPW) - idx_pad.shape[0],), jnp.int32)])
    kern = pl.kernel(
        functools.partial(_embed_sum_body, active, embed),
        out_type=jax.ShapeDtypeStruct((_NC, embed), jnp.float32),
        mesh=mesh,
        scratch_types=[
            pltpu.VMEM((_RPW,), jnp.int32),
            pltpu.VMEM((_RPW, embed), jnp.float32),
            pltpu.VMEM((embed,), jnp.float32),
            pltpu.VMEM((_NS, embed), jnp.float32),
            pltpu.VMEM_SHARED((_NS, embed), jnp.float32),
            pltpu.SemaphoreType.DMA,
        ],
    )
    return kern(padded, emb)


_KSC = 20480


def _tail_matvec_body(vstart, rpt, part_hbm, w_hbm, b_hbm, out_hbm,
                      pe_v, w_v, b_v, o_v):
    c = lax.axis_index("c")
    s = lax.axis_index("s")
    wid = s * _NC + c
    row0 = vstart + wid * rpt
    pltpu.sync_copy(part_hbm, pe_v)
    pltpu.sync_copy(b_hbm.at[pl.ds(row0, rpt)], b_v)
    pltpu.sync_copy(w_hbm.at[pl.ds(row0, rpt)], w_v)
    e = [pe_v[0, pl.ds(ch * _LANES, _LANES)] + pe_v[1, pl.ds(ch * _LANES, _LANES)]
         for ch in range(128 // _LANES)]
    lane = lax.iota(jnp.int32, _LANES)

    dnums = lax.GatherDimensionNumbers(
        offset_dims=(), collapsed_slice_dims=(0,), start_index_map=(0,))

    def hsum_splat(v):
        for k in (1, 2, 4, 8):
            perm = lax.gather(v, (lane ^ k).reshape(_LANES, 1), dnums, (1,),
                              mode=lax.GatherScatterMode.PROMISE_IN_BOUNDS)
            v = v + perm
        return v

    def group_body(g, carry):
        acc = jnp.zeros((_LANES,), jnp.float32)
        for j in range(_LANES):
            r = g * _LANES + j
            v = w_v[r, pl.ds(0, _LANES)] * e[0]
            for ch in range(1, 128 // _LANES):
                v = v + w_v[r, pl.ds(ch * _LANES, _LANES)] * e[ch]
            acc = jnp.where(lane == j, hsum_splat(v), acc)
        o_v[pl.ds(g * _LANES, _LANES)] = acc + b_v[pl.ds(g * _LANES, _LANES)]
        return carry

    lax.fori_loop(0, rpt // _LANES, group_body, 0)
    pltpu.sync_copy(o_v, out_hbm.at[pl.ds(wid * rpt, rpt)])


def _tail_matvec_sc(partials, W, b):
    vocab, embed = W.shape
    assert embed == 128 and _KSC % _NW == 0
    rpt = _KSC // _NW
    vstart = vocab - _KSC
    mesh = plsc.VectorSubcoreMesh(
        core_axis_name="c", subcore_axis_name="s",
        num_cores=_NC, num_subcores=_NS)
    kern = pl.kernel(
        functools.partial(_tail_matvec_body, vstart, rpt),
        out_type=jax.ShapeDtypeStruct((_KSC,), jnp.float32),
        mesh=mesh,
        scratch_types=[
            pltpu.VMEM((_NC, embed), jnp.float32),
            pltpu.VMEM((rpt, embed), jnp.float32),
            pltpu.VMEM((rpt,), jnp.float32),
            pltpu.VMEM((rpt,), jnp.float32),
        ],
    )
    return kern(partials, W, b)


_TILE = 16384


def _matvec_body(e_ref, w_ref, b_ref, o_ref):
    e = e_ref[0:1, :] + e_ref[1:2, :]
    o_ref[...] = jax.lax.dot_general(
        e, w_ref[...],
        dimension_numbers=(((1,), (1,)), ((), ())),
        preferred_element_type=jnp.float32) + b_ref[...]


def _matvec_tc(partials, W, b2, width):
    vocab, embed = W.shape
    grid = (width + _TILE - 1) // _TILE
    return pl.pallas_call(
        _matvec_body,
        grid=(grid,),
        in_specs=[
            pl.BlockSpec((_NC, embed), lambda i: (0, 0)),
            pl.BlockSpec((_TILE, embed), lambda i: (i, 0)),
            pl.BlockSpec((1, _TILE), lambda i: (0, i)),
        ],
        out_specs=pl.BlockSpec((1, _TILE), lambda i: (0, i)),
        out_shape=jax.ShapeDtypeStruct((1, width), jnp.float32),
    )(partials, W, b2)


def kernel(inputs, emb, W, b):
    idx = inputs.astype(jnp.int32)
    assert idx.shape[0] % _RPW == 0
    partials = _embed_sum_sc(idx, emb)
    out_sc = _tail_matvec_sc(partials, W, b)
    out_tc = _matvec_tc(partials, W, b.reshape(1, -1), W.shape[0] - _KSC)
    return jnp.concatenate([out_tc, out_sc.reshape(1, -1)], axis=1)


# speedup vs baseline: 1.0578x; 1.0578x over previous
import functools

import jax
import jax.numpy as jnp
from jax import lax
from jax.experimental import pallas as pl
from jax.experimental.pallas import tpu as pltpu
from jax.experimental.pallas import tpu_sc as plsc

_NC = 2
_NS = 16
_LANES = 16
_SPLIT = 96


def _embed_sum_body(ctx, embed, idx_hbm, emb_hbm, out_hbm,
                    idx_v, rows_v, acc_v, sem):
    c = lax.axis_index("c")
    s = lax.axis_index("s")
    nch = embed // _LANES

    def gather_sum(offset, nrows):
        pltpu.sync_copy(idx_hbm.at[pl.ds(offset, nrows)], idx_v.at[pl.ds(0, nrows)])
        pltpu.async_copy(emb_hbm.at[idx_v.at[pl.ds(0, nrows)]],
                         rows_v.at[pl.ds(0, nrows)], sem).wait()
        for ch in range(nch):
            acc_v[pl.ds(ch * _LANES, _LANES)] = rows_v[0, pl.ds(ch * _LANES, _LANES)]

        def row_body(j, carry):
            for ch in range(nch):
                sl = pl.ds(ch * _LANES, _LANES)
                acc_v[sl] = acc_v[sl] + rows_v[j, sl]
            return carry

        lax.fori_loop(1, nrows, row_body, 0)
        pltpu.sync_copy(acc_v, out_hbm.at[c])

    @pl.when(jnp.logical_and(s == 0, c == 0))
    def _():
        gather_sum(0, _SPLIT)

    @pl.when(jnp.logical_and(s == 0, c == 1))
    def _():
        gather_sum(_SPLIT, ctx - _SPLIT)


def _embed_sum_sc(idx, emb):
    embed = emb.shape[1]
    ctx = idx.shape[0]
    nmax = max(_SPLIT, ctx - _SPLIT)
    mesh = plsc.VectorSubcoreMesh(
        core_axis_name="c", subcore_axis_name="s",
        num_cores=_NC, num_subcores=_NS)
    kern = pl.kernel(
        functools.partial(_embed_sum_body, ctx, embed),
        out_type=jax.ShapeDtypeStruct((_NC, embed), jnp.float32),
        mesh=mesh,
        scratch_types=[
            pltpu.VMEM((nmax,), jnp.int32),
            pltpu.VMEM((nmax, embed), jnp.float32),
            pltpu.VMEM((embed,), jnp.float32),
            pltpu.SemaphoreType.DMA,
        ],
    )
    return kern(idx, emb)


_TILE = 16384


def _matvec_body(e_ref, w_ref, b_ref, o_ref):
    e = e_ref[0:1, :] + e_ref[1:2, :]
    o_ref[...] = jax.lax.dot_general(
        e, w_ref[...],
        dimension_numbers=(((1,), (1,)), ((), ())),
        preferred_element_type=jnp.float32) + b_ref[...].reshape(1, -1)


def _matvec_tc(partials, W, b):
    vocab, embed = W.shape
    grid = (vocab + _TILE - 1) // _TILE
    return pl.pallas_call(
        _matvec_body,
        grid=(grid,),
        in_specs=[
            pl.BlockSpec((_NC, embed), lambda i: (0, 0)),
            pl.BlockSpec((_TILE, embed), lambda i: (i, 0)),
            pl.BlockSpec((_TILE,), lambda i: (i,)),
        ],
        out_specs=pl.BlockSpec((1, _TILE), lambda i: (0, i)),
        out_shape=jax.ShapeDtypeStruct((1, vocab), jnp.float32),
    )(partials, W, b)


def kernel(inputs, emb, W, b):
    idx = inputs.astype(jnp.int32)
    partials = _embed_sum_sc(idx, emb)
    return _matvec_tc(partials, W, b)


# speedup vs baseline: 1.1637x; 1.1001x over previous
import functools

import jax
import jax.numpy as jnp
from jax import lax
from jax.experimental import pallas as pl
from jax.experimental.pallas import tpu as pltpu
from jax.experimental.pallas import tpu_sc as plsc

_LANES = 16


def _embed_sum_body(ctx, embed, idx_hbm, emb_hbm, out_hbm, idx_v, rows_v, acc_v, sem):
    nch = embed // _LANES
    pltpu.sync_copy(idx_hbm, idx_v)
    half = (ctx // 2 + 7) // 8 * 8
    cp0 = pltpu.async_copy(emb_hbm.at[idx_v.at[pl.ds(0, half)]],
                           rows_v.at[pl.ds(0, half)], sem)
    cp1 = pltpu.async_copy(emb_hbm.at[idx_v.at[pl.ds(half, ctx - half)]],
                           rows_v.at[pl.ds(half, ctx - half)], sem)
    cp0.wait()
    cp1.wait()

    def row_body(j, acc):
        return tuple(acc[ch] + rows_v[j, pl.ds(ch * _LANES, _LANES)]
                     for ch in range(nch))

    acc0 = tuple(rows_v[0, pl.ds(ch * _LANES, _LANES)] for ch in range(nch))
    acc = lax.fori_loop(1, ctx, row_body, acc0)
    for ch in range(nch):
        acc_v[pl.ds(ch * _LANES, _LANES)] = acc[ch]
    pltpu.sync_copy(acc_v, out_hbm.at[0])


def _embed_sum_sc(idx, emb):
    embed = emb.shape[1]
    ctx = idx.shape[0]
    mesh = plsc.VectorSubcoreMesh(
        core_axis_name="c", subcore_axis_name="s",
        num_cores=1, num_subcores=1)
    kern = pl.kernel(
        functools.partial(_embed_sum_body, ctx, embed),
        out_type=jax.ShapeDtypeStruct((1, embed), jnp.float32),
        mesh=mesh,
        scratch_types=[
            pltpu.VMEM((ctx,), jnp.int32),
            pltpu.VMEM((ctx, embed), jnp.float32),
            pltpu.VMEM((embed,), jnp.float32),
            pltpu.SemaphoreType.DMA,
        ],
    )
    return kern(idx, emb)


_TILE = 16384


def _matvec_body(e_ref, w_ref, b_ref, o_ref):
    o_ref[...] = jax.lax.dot_general(
        e_ref[...], w_ref[...],
        dimension_numbers=(((1,), (1,)), ((), ())),
        preferred_element_type=jnp.float32) + b_ref[...].reshape(1, -1)


def _matvec_tc(e, W, b):
    vocab, embed = W.shape
    grid = (vocab + _TILE - 1) // _TILE
    return pl.pallas_call(
        _matvec_body,
        grid=(grid,),
        in_specs=[
            pl.BlockSpec((1, embed), lambda i: (0, 0)),
            pl.BlockSpec((_TILE, embed), lambda i: (i, 0)),
            pl.BlockSpec((_TILE,), lambda i: (i,)),
        ],
        out_specs=pl.BlockSpec((1, _TILE), lambda i: (0, i)),
        out_shape=jax.ShapeDtypeStruct((1, vocab), jnp.float32),
    )(e, W, b)


def kernel(inputs, emb, W, b):
    idx = inputs.astype(jnp.int32)
    e = _embed_sum_sc(idx, emb)
    return _matvec_tc(e, W, b)
